# Initial kernel scaffold; baseline (speedup 1.0000x reference)
import jax
import jax.numpy as jnp
from jax.experimental import pallas as pl


def kernel(x6, x4, x12, weight, gamma, beta):
    raise NotImplementedError("write your pallas kernel here")


# speedup vs baseline: 5.8924x; 5.8924x over previous
import functools

import jax
import jax.numpy as jnp
from jax import lax
from jax.experimental import pallas as pl
from jax.experimental.pallas import tpu as pltpu

SCALE = 0.448
SHIFT = -0.08799999999999997
EPS = 1e-3


def _assemble_patches(b_ref, ho, wp):
    taps = [
        b_ref[(kh % 2) * 7 + kw, c, (kh // 2):(kh // 2) + ho, :]
        for c in range(3) for kh in range(7) for kw in range(7)
    ]
    p = jnp.stack(taps, axis=0)
    return p.reshape(147, ho * wp)


def _stats_kernel(b_ref, w_ref, s_ref, ss_ref, *, ho, wp):
    p = _assemble_patches(b_ref, ho, wp)
    y = jnp.dot(w_ref[...], p, preferred_element_type=jnp.float32)
    s_ref[...] = jnp.sum(y, axis=1, keepdims=True)
    ss_ref[...] = jnp.sum(y * y, axis=1, keepdims=True)


def _apply_kernel(b_ref, w_ref, s_ref, ss_ref, g_ref, be_ref, o_ref,
                  *, ho, wo, wp, inv_count):
    mean = jnp.sum(s_ref[...], axis=0) * inv_count
    var = jnp.sum(ss_ref[...], axis=0) * inv_count - mean * mean
    scale = g_ref[...] * lax.rsqrt(var + EPS)
    bias = be_ref[...] - mean * scale
    p = _assemble_patches(b_ref, ho, wp)
    y = jnp.dot(w_ref[...], p, preferred_element_type=jnp.float32)
    y = y.reshape(64, ho, wp)
    o_ref[...] = (y * scale[:, :, None] + bias[:, :, None])[:, :, :wo]


def kernel(x6, x4, x12, weight, gamma, beta):
    n, _, h, w = x6.shape
    cout = weight.shape[0]
    ho, wo = h // 2, w // 2
    wp = -(-wo // 128) * 128

    x13 = jnp.concatenate([x4, x6 * SCALE + SHIFT, x12], axis=1)
    xp = jnp.pad(x13, ((0, 0), (0, 0), (3, 3), (3, 3)))
    slabs = []
    for pr in range(2):
        for kw in range(7):
            slabs.append(lax.slice(
                xp, (0, 0, pr, kw),
                (n, 3, pr + 2 * (ho + 2) + 1, kw + 2 * (wo - 1) + 1),
                (1, 1, 2, 2)))
    ball = jnp.stack(slabs, axis=1)
    ball = jnp.pad(ball, ((0, 0),) * 4 + ((0, wp - wo),))
    ball = ball.astype(jnp.bfloat16)
    w2 = weight.reshape(cout, 147).astype(jnp.bfloat16)

    slab_spec = pl.BlockSpec((None, 14, 3, ho + 3, wp),
                             lambda i: (i, 0, 0, 0, 0))
    w_spec = pl.BlockSpec((cout, 147), lambda i: (0, 0))
    stat_out_spec = pl.BlockSpec((None, cout, 1), lambda i: (i, 0, 0))

    s, ss = pl.pallas_call(
        functools.partial(_stats_kernel, ho=ho, wp=wp),
        out_shape=(jax.ShapeDtypeStruct((n, cout, 1), jnp.float32),
                   jax.ShapeDtypeStruct((n, cout, 1), jnp.float32)),
        grid=(n,),
        in_specs=[slab_spec, w_spec],
        out_specs=(stat_out_spec, stat_out_spec),
        compiler_params=pltpu.CompilerParams(
            dimension_semantics=("parallel",),
            vmem_limit_bytes=100 * 1024 * 1024,
        ),
    )(ball, w2)

    g2 = gamma.reshape(cout, 1).astype(jnp.float32)
    b2 = beta.reshape(cout, 1).astype(jnp.float32)
    stat_in_spec = pl.BlockSpec((n, cout, 1), lambda i: (0, 0, 0))
    gb_spec = pl.BlockSpec((cout, 1), lambda i: (0, 0))
    out = pl.pallas_call(
        functools.partial(_apply_kernel, ho=ho, wo=wo, wp=wp,
                          inv_count=1.0 / float(n * ho * wo)),
        out_shape=jax.ShapeDtypeStruct((n, cout, ho, wo), jnp.float32),
        grid=(n,),
        in_specs=[slab_spec, w_spec, stat_in_spec, stat_in_spec,
                  gb_spec, gb_spec],
        out_specs=pl.BlockSpec((None, cout, ho, wo), lambda i: (i, 0, 0, 0)),
        compiler_params=pltpu.CompilerParams(
            dimension_semantics=("parallel",),
            vmem_limit_bytes=100 * 1024 * 1024,
        ),
    )(ball, w2, s, ss, g2, b2)
    return out
